# baseline (speedup 1.0000x reference)
import functools

import jax
import jax.numpy as jnp
from jax import lax
from jax.experimental import pallas as pl
from jax.experimental.pallas import tpu as pltpu
from jax.experimental.pallas import tpu_sc as plsc

NR_CATE = 26
CATE_PAD = 32
BATCH = 16384
NR_FEAT = 128

NC = 2
NS = 16
L = 16
NW = NC * NS
BPW = BATCH // NW
CHUNK = 128
NCH = BPW // CHUNK


def kernel(x, label):
    mesh = plsc.VectorSubcoreMesh(core_axis_name="c", subcore_axis_name="s")

    @functools.partial(
        pl.kernel,
        mesh=mesh,
        out_type=jax.ShapeDtypeStruct((BATCH, NR_FEAT), jnp.float32),
        compiler_params=pltpu.CompilerParams(use_tc_tiling_on_sc=False),
        scratch_types=[
            pltpu.VMEM((BPW,), jnp.int32),
            pltpu.VMEM((NCH, CHUNK), jnp.int32),
            pltpu.VMEM((BPW, NR_FEAT), jnp.float32),
            pltpu.SemaphoreType.DMA,
        ],
    )
    def k(x_hbm, label_hbm, out_hbm, label_v, idx_v, rows_v, sem):
        wid = lax.axis_index("s") * NC + lax.axis_index("c")
        base = wid * BPW
        pltpu.sync_copy(label_hbm.at[pl.ds(base, BPW)], label_v)
        lane = lax.iota(jnp.int32, L)
        table = x_hbm.at[0]
        copies = []
        for g in range(BPW // L):
            off = g * L
            lab = label_v[pl.ds(off, L)]
            idxv = (base + off + lane) * CATE_PAD + lab
            copies.append(
                pltpu.async_copy(
                    table.at[idxv], rows_v.at[pl.ds(off, L)], sem
                )
            )
        for cp in copies:
            cp.wait()
        pltpu.sync_copy(rows_v, out_hbm.at[pl.ds(base, BPW)])

    return k(x, label)


# speedup vs baseline: 1.9367x; 1.9367x over previous
import functools

import jax
import jax.numpy as jnp
from jax import lax
from jax.experimental import pallas as pl
from jax.experimental.pallas import tpu as pltpu
from jax.experimental.pallas import tpu_sc as plsc

NR_CATE = 26
CATE_PAD = 32
BATCH = 16384
NR_FEAT = 128

NC = 2
NS = 16
L = 16
NW = NC * NS
BPW = BATCH // NW
CHUNK = 128
NCH = BPW // CHUNK


def kernel(x, label):
    mesh = plsc.VectorSubcoreMesh(core_axis_name="c", subcore_axis_name="s")

    @functools.partial(
        pl.kernel,
        mesh=mesh,
        out_type=jax.ShapeDtypeStruct((BATCH, NR_FEAT), jnp.float32),
        scratch_types=[
            pltpu.VMEM((BPW,), jnp.int32),
            pltpu.VMEM((NCH, CHUNK), jnp.int32),
            pltpu.VMEM((BPW, NR_FEAT), jnp.float32),
            pltpu.SemaphoreType.DMA,
            pltpu.SemaphoreType.DMA,
            pltpu.SemaphoreType.DMA,
            pltpu.SemaphoreType.DMA,
        ],
    )
    def k(x_hbm, label_hbm, out_hbm, label_v, idx_v, rows_v, s0, s1, s2, s3):
        wid = lax.axis_index("s") * NC + lax.axis_index("c")
        base = wid * BPW
        sems = (s0, s1, s2, s3)
        pltpu.sync_copy(label_hbm.at[pl.ds(base, BPW)], label_v)
        lane = lax.iota(jnp.int32, L)
        table = x_hbm.at[0]
        copies = []
        for g in range(BPW // L):
            off = g * L
            lab = label_v[pl.ds(off, L)]
            idxv = (base + off + lane) * CATE_PAD + lab
            copies.append(
                pltpu.async_copy(
                    table.at[idxv], rows_v.at[pl.ds(off, L)], sems[g % 4]
                )
            )
        for cp in copies:
            cp.wait()
        pltpu.sync_copy(rows_v, out_hbm.at[pl.ds(base, BPW)])

    return k(x, label)


# speedup vs baseline: 1.9371x; 1.0002x over previous
import functools

import jax
import jax.numpy as jnp
from jax import lax
from jax.experimental import pallas as pl
from jax.experimental.pallas import tpu as pltpu
from jax.experimental.pallas import tpu_sc as plsc

NR_CATE = 26
CATE_PAD = 32
BATCH = 16384
NR_FEAT = 128

NC = 2
NS = 16
L = 16
NW = NC * NS
BPW = BATCH // NW
CHUNK = 128
NCH = BPW // CHUNK


def kernel(x, label):
    mesh = plsc.VectorSubcoreMesh(core_axis_name="c", subcore_axis_name="s")

    @functools.partial(
        pl.kernel,
        mesh=mesh,
        out_type=jax.ShapeDtypeStruct((BATCH, NR_FEAT), jnp.float32),
        scratch_types=[
            pltpu.VMEM((BPW,), jnp.int32),
            pltpu.VMEM((NCH, CHUNK), jnp.int32),
            pltpu.VMEM((BPW, NR_FEAT), jnp.float32),
            pltpu.SemaphoreType.DMA,
            pltpu.SemaphoreType.DMA,
            pltpu.SemaphoreType.DMA,
            pltpu.SemaphoreType.DMA,
            pltpu.SemaphoreType.DMA,
            pltpu.SemaphoreType.DMA,
            pltpu.SemaphoreType.DMA,
            pltpu.SemaphoreType.DMA,
            pltpu.SemaphoreType.DMA,
        ],
    )
    def k(x_hbm, label_hbm, out_hbm, label_v, idx_v, rows_v,
          s0, s1, s2, s3, s4, s5, s6, s7, ssem):
        wid = lax.axis_index("s") * NC + lax.axis_index("c")
        base = wid * BPW
        sems = (s0, s1, s2, s3, s4, s5, s6, s7)
        pltpu.sync_copy(label_hbm.at[pl.ds(base, BPW)], label_v)
        lane = lax.iota(jnp.int32, L)
        table = x_hbm.at[0]
        gathers = []
        for g in range(BPW // L):
            off = g * L
            lab = label_v[pl.ds(off, L)]
            idxv = (base + off + lane) * CATE_PAD + lab
            gathers.append(
                pltpu.async_copy(
                    table.at[idxv], rows_v.at[pl.ds(off, L)], sems[g // 4]
                )
            )
        scatters = []
        for sg in range(8):
            for cp in gathers[sg * 4:(sg + 1) * 4]:
                cp.wait()
            off = sg * 4 * L
            scatters.append(
                pltpu.async_copy(
                    rows_v.at[pl.ds(off, 4 * L)],
                    out_hbm.at[pl.ds(base + off, 4 * L)],
                    ssem,
                )
            )
        for cp in scatters:
            cp.wait()

    return k(x, label)


# speedup vs baseline: 1.9420x; 1.0025x over previous
import functools

import jax
import jax.numpy as jnp
from jax import lax
from jax.experimental import pallas as pl
from jax.experimental.pallas import tpu as pltpu
from jax.experimental.pallas import tpu_sc as plsc

NR_CATE = 26
CATE_PAD = 32
BATCH = 16384
NR_FEAT = 128

NC = 2
NS = 16
L = 16
NW = NC * NS
BPW = BATCH // NW
CHUNK = 128
NCH = BPW // CHUNK


def kernel(x, label):
    mesh = plsc.VectorSubcoreMesh(core_axis_name="c", subcore_axis_name="s")

    @functools.partial(
        pl.kernel,
        mesh=mesh,
        out_type=jax.ShapeDtypeStruct((BATCH, NR_FEAT), jnp.float32),
        scratch_types=[
            pltpu.VMEM((BPW,), jnp.int32),
            pltpu.VMEM((NCH, CHUNK), jnp.int32),
            pltpu.VMEM((BPW, NR_FEAT), jnp.float32),
            pltpu.SemaphoreType.DMA,
            pltpu.SemaphoreType.DMA,
            pltpu.SemaphoreType.DMA,
            pltpu.SemaphoreType.DMA,
        ],
    )
    def k(x_hbm, label_hbm, out_hbm, label_v, idx_v, rows_v, s0, s1, s2, s3):
        wid = lax.axis_index("s") * NC + lax.axis_index("c")
        base = wid * BPW
        sems = (s0, s1, s2, s3)
        pltpu.sync_copy(label_hbm.at[pl.ds(base, BPW)], label_v)
        lane = lax.iota(jnp.int32, L)
        table = x_hbm.at[0]
        copies = []
        for g in range(BPW // L):
            off = g * L
            lab = label_v[pl.ds(off, L)]
            idxv = (base + off + lane) * CATE_PAD + lab
            copies.append(
                pltpu.async_copy(
                    table.at[idxv], rows_v.at[pl.ds(off, L)], sems[g % 4]
                )
            )
        for cp in copies:
            cp.wait()
        pltpu.sync_copy(rows_v, out_hbm.at[pl.ds(base, BPW)])

    return k(x, label)
